# Initial kernel scaffold; baseline (speedup 1.0000x reference)
import functools

import jax
import jax.numpy as jnp
from jax import lax
from jax.experimental import pallas as pl
from jax.experimental.pallas import tpu as pltpu
from jax.experimental.pallas import tpu_sc as plsc

_N_IN = 100000
_N_OUT = 50000
_K = 8
_E = 12500
_C = 32
_EP = 12800
_TOT = _K * _EP
_NW = 32
_RPW = _TOT // _NW
_CH = 128
_CHUNKS = _RPW // _CH
_ACC_ROWS = 51200

_mesh = plsc.VectorSubcoreMesh(core_axis_name="c", subcore_axis_name="s")


@functools.partial(
    pl.kernel,
    mesh=_mesh,
    out_type=jax.ShapeDtypeStruct((_TOT, _C), jnp.float32),
    scratch_types=[
        pltpu.VMEM((_CHUNKS, _CH), jnp.int32),
        pltpu.VMEM((_CH, _C), jnp.float32),
        pltpu.SemaphoreType.DMA,
    ],
)
def _sc_gather(feats_hbm, idx_hbm, out_hbm, idx_v, buf, sem):
    wid = lax.axis_index("s") * 2 + lax.axis_index("c")
    pltpu.sync_copy(idx_hbm.at[wid], idx_v)
    base = wid * _RPW

    def body(j, carry):
        pltpu.async_copy(feats_hbm.at[idx_v.at[j]], buf, sem).wait()
        pltpu.sync_copy(buf, out_hbm.at[pl.ds(base + j * _CH, _CH)])
        return carry

    lax.fori_loop(0, _CHUNKS, body, 0)


@functools.partial(
    pl.kernel,
    mesh=_mesh,
    out_type=jax.ShapeDtypeStruct((2, _N_OUT, _C), jnp.float32),
    scratch_types=[
        pltpu.VMEM((_CHUNKS, _CH), jnp.int32),
        pltpu.VMEM((_CH, _C), jnp.float32),
        pltpu.VMEM_SHARED((_ACC_ROWS, _C), jnp.float32),
        pltpu.SemaphoreType.DMA,
    ],
)
def _sc_scatter(msg_hbm, oidx_hbm, part_hbm, idx_v, buf, acc, sem):
    cid = lax.axis_index("c")
    sid = lax.axis_index("s")
    wid = sid * 2 + cid
    stripe = sid * _RPW

    zeros16 = jnp.zeros((16,), jnp.float32)
    for i in range(_CH):
        buf[i, pl.ds(0, 16)] = zeros16
        buf[i, pl.ds(16, 16)] = zeros16

    def zchunk(q, carry):
        pltpu.sync_copy(buf, acc.at[pl.ds(stripe + q * _CH, _CH)])
        return carry

    lax.fori_loop(0, _CHUNKS, zchunk, 0)
    plsc.subcore_barrier()

    pltpu.sync_copy(oidx_hbm.at[wid], idx_v)
    rbase = wid * _RPW

    def body(j, carry):
        pltpu.sync_copy(msg_hbm.at[pl.ds(rbase + j * _CH, _CH)], buf)
        pltpu.sync_copy(buf, acc.at[idx_v.at[j]], add=True)
        return carry

    lax.fori_loop(0, _CHUNKS, body, 0)
    plsc.subcore_barrier()

    def wchunk(q, carry):
        wb = stripe + q * 400

        @pl.when(wb < _N_OUT)
        def _():
            pltpu.sync_copy(acc.at[pl.ds(wb, 400)], part_hbm.at[cid, pl.ds(wb, 400)])

        return carry

    lax.fori_loop(0, 8, wchunk, 0)


def _mm_body(g_ref, w_ref, o_ref):
    o_ref[0] = jnp.dot(g_ref[0], w_ref[0], preferred_element_type=jnp.float32)


_BM = 1600


def _tc_matmul(gathered, W):
    return pl.pallas_call(
        _mm_body,
        grid=(_K, _EP // _BM),
        in_specs=[
            pl.BlockSpec((1, _BM, _C), lambda k, i: (k, i, 0)),
            pl.BlockSpec((1, _C, _C), lambda k, i: (k, 0, 0)),
        ],
        out_specs=pl.BlockSpec((1, _BM, _C), lambda k, i: (k, i, 0)),
        out_shape=jax.ShapeDtypeStruct((_K, _EP, _C), jnp.float32),
    )(gathered, W)


def _add_body(p_ref, o_ref):
    o_ref[...] = p_ref[0] + p_ref[1]


def _tc_add(partials):
    rows = _N_OUT * _C // 128
    bm = 2500
    return pl.pallas_call(
        _add_body,
        grid=(rows // bm,),
        in_specs=[pl.BlockSpec((2, bm, 128), lambda i: (0, i, 0))],
        out_specs=pl.BlockSpec((bm, 128), lambda i: (i, 0)),
        out_shape=jax.ShapeDtypeStruct((rows, 128), jnp.float32),
    )(partials)


def kernel(feats, in_indices, out_indices, W):
    pad = _EP - _E
    in_p = jnp.pad(in_indices, ((0, 0), (0, pad)))
    out_p = jnp.pad(out_indices, ((0, 0), (0, pad)), constant_values=_N_OUT)
    idx_in = in_p.reshape(_NW, _CHUNKS, _CH)
    idx_out = out_p.reshape(_NW, _CHUNKS, _CH)
    gathered = _sc_gather(feats, idx_in)
    msg = _tc_matmul(gathered.reshape(_K, _EP, _C), W)
    partials = _sc_scatter(msg.reshape(_TOT, _C), idx_out)
    out = _tc_add(partials.reshape(2, _N_OUT * _C // 128, 128))
    return out.reshape(_N_OUT, _C)


# speedup vs baseline: 2.7475x; 2.7475x over previous
import functools

import jax
import jax.numpy as jnp
from jax import lax
from jax.experimental import pallas as pl
from jax.experimental.pallas import tpu as pltpu
from jax.experimental.pallas import tpu_sc as plsc

_N_IN = 100000
_N_OUT = 50000
_K = 8
_E = 12500
_C = 32
_EP = 12800
_TOT = _K * _EP
_NW = 32
_RPW = _TOT // _NW
_CH = 128
_CHUNKS = _RPW // _CH
_ACC_ROWS = 51200

_mesh = plsc.VectorSubcoreMesh(core_axis_name="c", subcore_axis_name="s")


@functools.partial(
    pl.kernel,
    mesh=_mesh,
    out_type=jax.ShapeDtypeStruct((_TOT, _C), jnp.float32),
    scratch_types=[
        pltpu.VMEM((_CHUNKS, _CH), jnp.int32),
        pltpu.VMEM((_CH, _C), jnp.float32),
        pltpu.SemaphoreType.DMA,
    ],
    compiler_params=pltpu.CompilerParams(use_tc_tiling_on_sc=False),
)
def _sc_gather(feats_hbm, idx_hbm, out_hbm, idx_v, buf, sem):
    wid = lax.axis_index("s") * 2 + lax.axis_index("c")
    pltpu.sync_copy(idx_hbm.at[wid], idx_v)
    base = wid * _RPW

    def body(j, carry):
        pltpu.async_copy(feats_hbm.at[idx_v.at[j]], buf, sem).wait()
        pltpu.sync_copy(buf, out_hbm.at[pl.ds(base + j * _CH, _CH)])
        return carry

    lax.fori_loop(0, _CHUNKS, body, 0)


@functools.partial(
    pl.kernel,
    mesh=_mesh,
    out_type=jax.ShapeDtypeStruct((2, _N_OUT, _C), jnp.float32),
    scratch_types=[
        pltpu.VMEM((_CHUNKS, _CH), jnp.int32),
        pltpu.VMEM((_CH, _C), jnp.float32),
        pltpu.VMEM_SHARED((_ACC_ROWS, _C), jnp.float32),
        pltpu.SemaphoreType.DMA,
    ],
    compiler_params=pltpu.CompilerParams(use_tc_tiling_on_sc=False),
)
def _sc_scatter(msg_hbm, oidx_hbm, part_hbm, idx_v, buf, acc, sem):
    cid = lax.axis_index("c")
    sid = lax.axis_index("s")
    wid = sid * 2 + cid
    stripe = sid * _RPW

    zeros16 = jnp.zeros((16,), jnp.float32)
    for i in range(_CH):
        buf[i, pl.ds(0, 16)] = zeros16
        buf[i, pl.ds(16, 16)] = zeros16

    def zchunk(q, carry):
        pltpu.sync_copy(buf, acc.at[pl.ds(stripe + q * _CH, _CH)])
        return carry

    lax.fori_loop(0, _CHUNKS, zchunk, 0)
    plsc.subcore_barrier()

    pltpu.sync_copy(oidx_hbm.at[wid], idx_v)
    rbase = wid * _RPW

    def body(j, carry):
        pltpu.sync_copy(msg_hbm.at[pl.ds(rbase + j * _CH, _CH)], buf)
        pltpu.sync_copy(buf, acc.at[idx_v.at[j]], add=True)
        return carry

    lax.fori_loop(0, _CHUNKS, body, 0)
    plsc.subcore_barrier()

    def wchunk(q, carry):
        wb = stripe + q * 400

        @pl.when(wb < _N_OUT)
        def _():
            pltpu.sync_copy(acc.at[pl.ds(wb, 400)], part_hbm.at[cid, pl.ds(wb, 400)])

        return carry

    lax.fori_loop(0, 8, wchunk, 0)


def _mm_body(g_ref, w_ref, o_ref):
    o_ref[0] = jnp.dot(g_ref[0], w_ref[0], preferred_element_type=jnp.float32)


_BM = 1600


def _tc_matmul(gathered, W):
    return pl.pallas_call(
        _mm_body,
        grid=(_K, _EP // _BM),
        in_specs=[
            pl.BlockSpec((1, _BM, _C), lambda k, i: (k, i, 0)),
            pl.BlockSpec((1, _C, _C), lambda k, i: (k, 0, 0)),
        ],
        out_specs=pl.BlockSpec((1, _BM, _C), lambda k, i: (k, i, 0)),
        out_shape=jax.ShapeDtypeStruct((_K, _EP, _C), jnp.float32),
    )(gathered, W)


def _add_body(p_ref, o_ref):
    o_ref[...] = p_ref[0] + p_ref[1]


def _tc_add(partials):
    return pl.pallas_call(
        _add_body,
        grid=(10,),
        in_specs=[pl.BlockSpec((2, 10, 125, 128), lambda i: (0, i, 0, 0))],
        out_specs=pl.BlockSpec((10, 125, 128), lambda i: (i, 0, 0)),
        out_shape=jax.ShapeDtypeStruct((100, 125, 128), jnp.float32),
    )(partials)


def kernel(feats, in_indices, out_indices, W):
    pad = _EP - _E
    in_p = jnp.pad(in_indices, ((0, 0), (0, pad)))
    out_p = jnp.pad(out_indices, ((0, 0), (0, pad)), constant_values=_N_OUT)
    idx_in = in_p.reshape(_NW, _CHUNKS, _CH)
    idx_out = out_p.reshape(_NW, _CHUNKS, _CH)
    gathered = _sc_gather(feats, idx_in)
    msg = _tc_matmul(gathered.reshape(_K, _EP, _C), W)
    partials = _sc_scatter(msg.reshape(_TOT, _C), idx_out)
    out = _tc_add(partials.reshape(2, 100, 125, 128))
    return out.reshape(_N_OUT, _C)
